# Initial kernel scaffold; baseline (speedup 1.0000x reference)
import jax
import jax.numpy as jnp
from jax.experimental import pallas as pl


def kernel(x, pos, token_embed, pos_embed):
    raise NotImplementedError("write your pallas kernel here")


# speedup vs baseline: 1.9010x; 1.9010x over previous
import functools

import jax
import jax.numpy as jnp
from jax import lax
from jax.experimental import pallas as pl
from jax.experimental.pallas import tpu as pltpu
from jax.experimental.pallas import tpu_sc as plsc

DIM = 64
LANES = 16
NUM_CORES = 2
NUM_SUBCORES = 16
NUM_WORKERS = NUM_CORES * NUM_SUBCORES

CHUNK = 512
IDX_SUB = 128
SUBS = CHUNK // IDX_SUB


def _sc_embed_add(n_rows):
    assert n_rows % (NUM_WORKERS * CHUNK) == 0
    rows_per_w = n_rows // NUM_WORKERS
    chunks = rows_per_w // CHUNK
    idx_rows_per_w = rows_per_w // IDX_SUB

    mesh = plsc.VectorSubcoreMesh(
        core_axis_name="c", subcore_axis_name="s",
        num_cores=NUM_CORES, num_subcores=NUM_SUBCORES)

    @functools.partial(
        pl.kernel,
        out_type=jax.ShapeDtypeStruct((n_rows, DIM), jnp.float32),
        mesh=mesh,
        compiler_params=pltpu.CompilerParams(use_tc_tiling_on_sc=False),
        scratch_types=[
            pltpu.VMEM((idx_rows_per_w, IDX_SUB), jnp.int32),
            pltpu.VMEM((idx_rows_per_w, IDX_SUB), jnp.int32),
            pltpu.VMEM((CHUNK, DIM), jnp.float32),
            pltpu.VMEM((CHUNK, DIM), jnp.float32),
            pltpu.SemaphoreType.DMA,
        ],
    )
    def k(x_hbm, pos_hbm, tok_hbm, posemb_hbm, out_hbm,
          idx_t, idx_p, rows_t, rows_p, sem):
        wid = lax.axis_index("s") * NUM_CORES + lax.axis_index("c")
        base_row = wid * rows_per_w
        idx_base = wid * idx_rows_per_w
        pltpu.sync_copy(x_hbm.at[pl.ds(idx_base, idx_rows_per_w)], idx_t)
        pltpu.sync_copy(pos_hbm.at[pl.ds(idx_base, idx_rows_per_w)], idx_p)

        def chunk_body(c, _):
            off = base_row + c * CHUNK
            cps = []
            for j in range(SUBS):
                row = c * SUBS + j
                dst = rows_t.at[pl.ds(j * IDX_SUB, IDX_SUB)]
                cps.append(pltpu.async_copy(tok_hbm.at[idx_t.at[row]], dst,
                                            sem))
                dst = rows_p.at[pl.ds(j * IDX_SUB, IDX_SUB)]
                cps.append(pltpu.async_copy(posemb_hbm.at[idx_p.at[row]], dst,
                                            sem))
            for cp in cps:
                cp.wait()

            def add_body(r, _):
                for q in range(DIM // LANES):
                    sl = pl.ds(q * LANES, LANES)
                    rows_t[r, sl] = rows_t[r, sl] + rows_p[r, sl]
                return _
            lax.fori_loop(0, CHUNK, add_body, 0, unroll=2)

            pltpu.sync_copy(rows_t, out_hbm.at[pl.ds(off, CHUNK)])
            return _

        lax.fori_loop(0, chunks, chunk_body, 0)

    return k


def kernel(x, pos, token_embed, pos_embed):
    b, l = x.shape
    n = b * l
    x_flat = x.reshape(n // IDX_SUB, IDX_SUB)
    pos_flat = pos.reshape(n // IDX_SUB, IDX_SUB)
    out = _sc_embed_add(n)(x_flat, pos_flat, token_embed, pos_embed)
    return out.reshape(b, l, DIM)


# speedup vs baseline: 2.0316x; 1.0687x over previous
import functools

import jax
import jax.numpy as jnp
from jax import lax
from jax.experimental import pallas as pl
from jax.experimental.pallas import tpu as pltpu
from jax.experimental.pallas import tpu_sc as plsc

DIM = 64
LANES = 16
NUM_CORES = 2
NUM_SUBCORES = 16
NUM_WORKERS = NUM_CORES * NUM_SUBCORES

CHUNK = 512
IDX_SUB = 128
SUBS = CHUNK // IDX_SUB


def _sc_embed_add(n_rows):
    assert n_rows % (NUM_WORKERS * CHUNK) == 0
    rows_per_w = n_rows // NUM_WORKERS
    chunks = rows_per_w // CHUNK
    idx_rows_per_w = rows_per_w // IDX_SUB

    mesh = plsc.VectorSubcoreMesh(
        core_axis_name="c", subcore_axis_name="s",
        num_cores=NUM_CORES, num_subcores=NUM_SUBCORES)

    @functools.partial(
        pl.kernel,
        out_type=jax.ShapeDtypeStruct((n_rows, DIM), jnp.float32),
        mesh=mesh,
        compiler_params=pltpu.CompilerParams(use_tc_tiling_on_sc=False),
        scratch_types=[
            pltpu.VMEM((idx_rows_per_w, IDX_SUB), jnp.int32),
            pltpu.VMEM((idx_rows_per_w, IDX_SUB), jnp.int32),
            pltpu.VMEM((CHUNK, DIM), jnp.float32),
            pltpu.VMEM((CHUNK, DIM), jnp.float32),
            pltpu.SemaphoreType.DMA,
        ],
    )
    def k(x_hbm, pos_hbm, tok_hbm, posemb_hbm, out_hbm,
          idx_t, idx_p, rows_t, rows_p, sem):
        wid = lax.axis_index("s") * NUM_CORES + lax.axis_index("c")
        base_row = wid * rows_per_w
        idx_base = wid * idx_rows_per_w
        pltpu.sync_copy(x_hbm.at[pl.ds(idx_base, idx_rows_per_w)], idx_t)
        pltpu.sync_copy(pos_hbm.at[pl.ds(idx_base, idx_rows_per_w)], idx_p)

        def chunk_body(c, _):
            off = base_row + c * CHUNK
            cps = []
            for j in range(SUBS):
                row = c * SUBS + j
                dst = rows_t.at[pl.ds(j * IDX_SUB, IDX_SUB)]
                cps.append(pltpu.async_copy(tok_hbm.at[idx_t.at[row]], dst,
                                            sem))
                dst = rows_p.at[pl.ds(j * IDX_SUB, IDX_SUB)]
                cps.append(pltpu.async_copy(posemb_hbm.at[idx_p.at[row]], dst,
                                            sem))
            for cp in cps:
                cp.wait()

            @plsc.parallel_loop(0, CHUNK, step=1, unroll=8)
            def add_body(r):
                for q in range(DIM // LANES):
                    sl = pl.ds(q * LANES, LANES)
                    rows_t[r, sl] = rows_t[r, sl] + rows_p[r, sl]

            pltpu.sync_copy(rows_t, out_hbm.at[pl.ds(off, CHUNK)])
            return _

        lax.fori_loop(0, chunks, chunk_body, 0)

    return k


def kernel(x, pos, token_embed, pos_embed):
    b, l = x.shape
    n = b * l
    x_flat = x.reshape(n // IDX_SUB, IDX_SUB)
    pos_flat = pos.reshape(n // IDX_SUB, IDX_SUB)
    out = _sc_embed_add(n)(x_flat, pos_flat, token_embed, pos_embed)
    return out.reshape(b, l, DIM)


# speedup vs baseline: 2.7531x; 1.3551x over previous
import functools

import jax
import jax.numpy as jnp
from jax import lax
from jax.experimental import pallas as pl
from jax.experimental.pallas import tpu as pltpu
from jax.experimental.pallas import tpu_sc as plsc

DIM = 64
LANES = 16
NUM_CORES = 2
NUM_SUBCORES = 16
NUM_WORKERS = NUM_CORES * NUM_SUBCORES

CHUNK = 128
NBUF = 3


def _sc_embed_add(n_rows, pos_vocab):
    assert n_rows % (NUM_WORKERS * CHUNK) == 0
    rows_per_w = n_rows // NUM_WORKERS
    chunks = rows_per_w // CHUNK
    assert chunks >= 5 and (chunks - 2) % NBUF == 0

    mesh = plsc.VectorSubcoreMesh(
        core_axis_name="c", subcore_axis_name="s",
        num_cores=NUM_CORES, num_subcores=NUM_SUBCORES)

    @functools.partial(
        pl.kernel,
        out_type=jax.ShapeDtypeStruct((n_rows, DIM), jnp.float32),
        mesh=mesh,
        compiler_params=pltpu.CompilerParams(use_tc_tiling_on_sc=False,
                                             needs_layout_passes=False),
        scratch_types=(
            [pltpu.VMEM((rows_per_w,), jnp.int32),
             pltpu.VMEM((rows_per_w,), jnp.int32),
             pltpu.VMEM((pos_vocab * DIM,), jnp.float32)]
            + [pltpu.VMEM((CHUNK, DIM), jnp.float32) for _ in range(NBUF)]
            + [pltpu.SemaphoreType.DMA for _ in range(2 * NBUF)]
        ),
    )
    def k(x_hbm, pos_hbm, tok_hbm, posemb_hbm, out_hbm,
          idx_t, idx_p, pos_tab, b0, b1, b2, g0, g1, g2, s0, s1, s2):
        wid = lax.axis_index("s") * NUM_CORES + lax.axis_index("c")
        base = wid * rows_per_w
        pltpu.sync_copy(x_hbm.at[pl.ds(base, rows_per_w)], idx_t)
        pltpu.sync_copy(pos_hbm.at[pl.ds(base, rows_per_w)], idx_p)
        pltpu.sync_copy(posemb_hbm, pos_tab)

        bufs = (b0, b1, b2)
        gsems = (g0, g1, g2)
        ssems = (s0, s1, s2)
        iota = lax.iota(jnp.int32, LANES)
        dnums = lax.GatherDimensionNumbers(
            offset_dims=(), collapsed_slice_dims=(0,), start_index_map=(0,))

        def g_desc(c, b):
            src = tok_hbm.at[idx_t.at[pl.ds(c * CHUNK, CHUNK)]]
            return pltpu.make_async_copy(src, bufs[b], gsems[b])

        def s_desc(c, b):
            dst = out_hbm.at[pl.ds(base + c * CHUNK, CHUNK)]
            return pltpu.make_async_copy(bufs[b], dst, ssems[b])

        zeros = jnp.zeros((LANES,), jnp.int32)

        def add_chunk(c, b):
            buf = bufs[b]

            @plsc.parallel_loop(0, CHUNK, step=1, unroll=2)
            def _row(r):
                pidx = plsc.load_gather(idx_p, [zeros + (c * CHUNK + r)])
                rbase = pidx * DIM
                for q in range(DIM // LANES):
                    addr = rbase + (iota + q * LANES)
                    pv = plsc.load_gather(pos_tab, [addr])
                    sl = pl.ds(q * LANES, LANES)
                    buf[r, sl] = buf[r, sl] + pv

        def step(j, b, first, last):
            nxt = (b + 1) % NBUF
            if not first:
                s_desc(j - 2, nxt).wait()
            if not last:
                g_desc(j + 1, nxt).start()
            g_desc(j, b).wait()
            add_chunk(j, b)
            s_desc(j, b).start()

        g_desc(0, 0).start()
        step(0, 0, first=True, last=False)
        step(1, 1, first=True, last=False)

        def body3(t, carry):
            for kk in range(NBUF):
                j = NBUF * t + (2 + kk)
                step(j, (2 + kk) % NBUF, first=False, last=False)
            return carry
        lax.fori_loop(0, (chunks - 2) // NBUF - 1, body3, 0)

        for j in range(chunks - NBUF, chunks):
            step(j, j % NBUF, first=False, last=(j == chunks - 1))
        s_desc(chunks - 2, (chunks - 2) % NBUF).wait()
        s_desc(chunks - 1, (chunks - 1) % NBUF).wait()

    return k


def kernel(x, pos, token_embed, pos_embed):
    b, l = x.shape
    n = b * l
    pos_vocab = pos_embed.shape[0]
    out = _sc_embed_add(n, pos_vocab)(
        x.reshape(n), pos.reshape(n), token_embed,
        pos_embed.reshape(pos_vocab * DIM))
    return out.reshape(b, l, DIM)
